# baseline (speedup 1.0000x reference)
import jax
import jax.numpy as jnp
from jax.experimental import pallas as pl
from jax.experimental.pallas import tpu as pltpu

F32 = jnp.float32
_VMEM_LIMIT = 56 * 1024 * 1024
_INTERPRET = False


def _cparams(*sems):
    return pltpu.CompilerParams(
        dimension_semantics=tuple(sems),
        vmem_limit_bytes=_VMEM_LIMIT,
    )


def _gemv_kernel(h_ref, w_ref, z_ref, *, relu):
    h = h_ref[...]
    if relu:
        h = jnp.maximum(h, 0.0)
    z_ref[...] = jax.lax.dot_general(
        h, w_ref[...], (((1,), (1,)), ((), ())),
        preferred_element_type=F32)


def _gemv(h, W, bj, relu):
    import functools
    J, K = W.shape
    return pl.pallas_call(
        functools.partial(_gemv_kernel, relu=relu),
        grid=(J // bj,),
        in_specs=[pl.BlockSpec((1, K), lambda j: (0, 0)),
                  pl.BlockSpec((bj, K), lambda j: (j, 0))],
        out_specs=pl.BlockSpec((1, bj), lambda j: (0, j)),
        out_shape=jax.ShapeDtypeStruct((1, J), F32),
        compiler_params=_cparams("parallel"),
        name="gemv",
        interpret=_INTERPRET,
    )(h, W)


def _transpose_kernel(w_ref, o_ref):
    o_ref[...] = w_ref[...].T


def _transpose(W, b=512):
    R, C = W.shape
    return pl.pallas_call(
        _transpose_kernel,
        grid=(R // b, C // b),
        in_specs=[pl.BlockSpec((b, b), lambda r, c: (r, c))],
        out_specs=pl.BlockSpec((b, b), lambda r, c: (c, r)),
        out_shape=jax.ShapeDtypeStruct((C, R), F32),
        compiler_params=_cparams("parallel", "arbitrary"),
        name="transpose",
        interpret=_INTERPRET,
    )(W)


def _diag_kernel(z1_ref, z2_ref, o1_ref, o2_ref, *, br, n):
    r = pl.program_id(0)
    rows = jax.lax.broadcasted_iota(jnp.int32, (br, n), 0) + r * br
    cols = jax.lax.broadcasted_iota(jnp.int32, (br, n), 1)
    eq = rows == cols
    o1_ref[...] = jnp.where(eq, (z1_ref[...] > 0).astype(F32), 0.0)
    o2_ref[...] = jnp.where(eq, (z2_ref[...] > 0).astype(F32), 0.0)


def _diag_pair(z1, z2, br=512):
    import functools
    n = z1.shape[1]
    out = jax.ShapeDtypeStruct((n, n), F32)
    return pl.pallas_call(
        functools.partial(_diag_kernel, br=br, n=n),
        grid=(n // br,),
        in_specs=[pl.BlockSpec((1, n), lambda r: (0, 0)),
                  pl.BlockSpec((1, n), lambda r: (0, 0))],
        out_specs=[pl.BlockSpec((br, n), lambda r: (r, 0)),
                   pl.BlockSpec((br, n), lambda r: (r, 0))],
        out_shape=[out, out],
        compiler_params=_cparams("parallel"),
        name="diag_pair",
        interpret=_INTERPRET,
    )(z1, z2)


def _mm1_kernel(a_ref, z_ref, b_ref, o_ref):
    scale = (z_ref[...] > 0).astype(F32)
    a = (a_ref[...] * scale).astype(jnp.bfloat16)
    b = b_ref[...].astype(jnp.bfloat16)
    o_ref[...] = jnp.dot(a, b, preferred_element_type=F32)


def _mm1(A, z, B, bi, bj):
    M, K = A.shape
    _, N = B.shape
    return pl.pallas_call(
        _mm1_kernel,
        grid=(M // bi, N // bj),
        in_specs=[pl.BlockSpec((bi, K), lambda i, j: (i, 0)),
                  pl.BlockSpec((1, K), lambda i, j: (0, 0)),
                  pl.BlockSpec((K, bj), lambda i, j: (0, j))],
        out_specs=pl.BlockSpec((bi, bj), lambda i, j: (i, j)),
        out_shape=jax.ShapeDtypeStruct((M, N), F32),
        compiler_params=_cparams("parallel", "arbitrary"),
        name="scaled_mm1",
        interpret=_INTERPRET,
    )(A, z, B)


def _mm2_kernel(a_ref, z_ref, b_ref, o_ref, eye_ref, *, bi, bl):
    i = pl.program_id(0)
    l = pl.program_id(1)
    scale = (z_ref[...] > 0).astype(F32)
    a = (a_ref[...] * scale).astype(jnp.bfloat16)
    b = b_ref[...].astype(jnp.bfloat16)
    o_ref[...] = jnp.dot(a, b, preferred_element_type=F32)
    rows = jax.lax.broadcasted_iota(jnp.int32, (bi, bl), 0) + i * bi
    cols = jax.lax.broadcasted_iota(jnp.int32, (bi, bl), 1) + l * bl
    eye_ref[...] = jnp.where(rows == cols, 1.0, 0.0).astype(F32)


def _mm2(A, z, B, bi, bl):
    import functools
    M, K = A.shape
    _, N = B.shape
    return pl.pallas_call(
        functools.partial(_mm2_kernel, bi=bi, bl=bl),
        grid=(M // bi, N // bl),
        in_specs=[pl.BlockSpec((bi, K), lambda i, l: (i, 0)),
                  pl.BlockSpec((1, K), lambda i, l: (0, 0)),
                  pl.BlockSpec((K, bl), lambda i, l: (0, l))],
        out_specs=[pl.BlockSpec((bi, bl), lambda i, l: (i, l)),
                   pl.BlockSpec((bi, bl), lambda i, l: (i, l))],
        out_shape=[jax.ShapeDtypeStruct((M, N), F32),
                   jax.ShapeDtypeStruct((M, N), F32)],
        compiler_params=_cparams("parallel", "arbitrary"),
        name="scaled_mm2_eye",
        interpret=_INTERPRET,
    )(A, z, B)


def kernel(x, W1, W2, W3):
    z1 = _gemv(x, W1, 512, relu=False)
    z2 = _gemv(z1, W2, 512, relu=True)
    out = _gemv(z2, W3, 512, relu=True)

    W1T = _transpose(W1)
    W2T = _transpose(W2)
    W3T = _transpose(W3)
    D1, D2 = _diag_pair(z1, z2)

    T1 = _mm1(W1T, z1, W2T, bi=1024, bj=256)
    DJM, EYE = _mm2(T1, z2, W3T, bi=1024, bl=256)

    return (out, DJM, W1T, D1, W2T, D2, W3T, EYE)


# speedup vs baseline: 1.2440x; 1.2440x over previous
import functools

import jax
import jax.numpy as jnp
from jax.experimental import pallas as pl
from jax.experimental.pallas import tpu as pltpu

F32 = jnp.float32
BF16 = jnp.bfloat16
_VMEM_LIMIT = 56 * 1024 * 1024
_INTERPRET = False


def _cparams(n):
    return pltpu.CompilerParams(
        dimension_semantics=("arbitrary",) * n,
        vmem_limit_bytes=_VMEM_LIMIT,
    )


def _gemv_kernel(h_ref, w_ref, z_ref, *, relu):
    h = h_ref[...]
    if relu:
        h = jnp.maximum(h, 0.0)
    z_ref[...] = jax.lax.dot_general(
        h, w_ref[...], (((1,), (1,)), ((), ())),
        preferred_element_type=F32)


def _gemv(h, W, bj, relu):
    J, K = W.shape
    return pl.pallas_call(
        functools.partial(_gemv_kernel, relu=relu),
        grid=(J // bj,),
        in_specs=[pl.BlockSpec((1, K), lambda j: (0, 0)),
                  pl.BlockSpec((bj, K), lambda j: (j, 0))],
        out_specs=pl.BlockSpec((1, bj), lambda j: (0, j)),
        out_shape=jax.ShapeDtypeStruct((1, J), F32),
        compiler_params=_cparams(1),
        name="gemv",
        interpret=_INTERPRET,
    )(h, W)


def _trans_kernel(w_ref, o_ref, ob_ref):
    wt = w_ref[...].T
    o_ref[...] = wt
    ob_ref[...] = wt.astype(BF16)


def _trans_scaled_kernel(w_ref, z_ref, o_ref, ob_ref):
    wt = w_ref[...].T
    o_ref[...] = wt
    ob_ref[...] = (wt * (z_ref[...] > 0).astype(F32)).astype(BF16)


def _transpose(W, z=None, b=1024):
    R, C = W.shape
    out = [jax.ShapeDtypeStruct((C, R), F32), jax.ShapeDtypeStruct((C, R), BF16)]
    out_specs = [pl.BlockSpec((b, b), lambda r, c: (c, r)),
                 pl.BlockSpec((b, b), lambda r, c: (c, r))]
    if z is None:
        return pl.pallas_call(
            _trans_kernel,
            grid=(R // b, C // b),
            in_specs=[pl.BlockSpec((b, b), lambda r, c: (r, c))],
            out_specs=out_specs,
            out_shape=out,
            compiler_params=_cparams(2),
            name="transpose",
            interpret=_INTERPRET,
        )(W)
    return pl.pallas_call(
        _trans_scaled_kernel,
        grid=(R // b, C // b),
        in_specs=[pl.BlockSpec((b, b), lambda r, c: (r, c)),
                  pl.BlockSpec((1, b), lambda r, c: (0, r))],
        out_specs=out_specs,
        out_shape=out,
        compiler_params=_cparams(2),
        name="transpose_scaled",
        interpret=_INTERPRET,
    )(W, z)


def _mm1_kernel(a_ref, b_ref, z1_ref, z2_ref, t1_ref, d1_ref, d2_ref,
                *, bi, bj):
    i = pl.program_id(0)
    j = pl.program_id(1)
    o = jnp.dot(a_ref[...], b_ref[...], preferred_element_type=F32)
    m2 = (z2_ref[...] > 0).astype(F32)
    t1_ref[...] = (o * m2).astype(BF16)
    rows = jax.lax.broadcasted_iota(jnp.int32, (2 * bi, bj), 0) + i * 2 * bi
    cols = jax.lax.broadcasted_iota(jnp.int32, (2 * bi, bj), 1) + j * bj
    eq = rows == cols
    m1 = (z1_ref[...] > 0).astype(F32)
    d1_ref[...] = jnp.where(eq, m1, 0.0)
    d2_ref[...] = jnp.where(eq, m2, 0.0)


def _mm1(A, B, z1, z2, bi=1024, bj=256):
    M, K = A.shape
    _, N = B.shape
    dshape = jax.ShapeDtypeStruct((K, N), F32)
    return pl.pallas_call(
        functools.partial(_mm1_kernel, bi=bi, bj=bj),
        grid=(M // bi, N // bj),
        in_specs=[pl.BlockSpec((bi, K), lambda i, j: (i, 0)),
                  pl.BlockSpec((K, bj), lambda i, j: (0, j)),
                  pl.BlockSpec((1, bj), lambda i, j: (0, j)),
                  pl.BlockSpec((1, bj), lambda i, j: (0, j))],
        out_specs=[pl.BlockSpec((bi, bj), lambda i, j: (i, j)),
                   pl.BlockSpec((2 * bi, bj), lambda i, j: (i, j)),
                   pl.BlockSpec((2 * bi, bj), lambda i, j: (i, j))],
        out_shape=[jax.ShapeDtypeStruct((M, N), BF16), dshape, dshape],
        compiler_params=_cparams(2),
        name="mm1_diag",
        interpret=_INTERPRET,
    )(A, B, z1, z2)


def _mm2_kernel(a_ref, b_ref, o_ref, eye_ref, *, m, bl):
    l = pl.program_id(0)
    o_ref[...] = jnp.dot(a_ref[...], b_ref[...], preferred_element_type=F32)
    rows = jax.lax.broadcasted_iota(jnp.int32, (m, bl), 0)
    cols = jax.lax.broadcasted_iota(jnp.int32, (m, bl), 1) + l * bl
    eye_ref[...] = jnp.where(rows == cols, 1.0, 0.0).astype(F32)


def _mm2(A, B, bl=256):
    M, K = A.shape
    _, N = B.shape
    oshape = jax.ShapeDtypeStruct((M, N), F32)
    return pl.pallas_call(
        functools.partial(_mm2_kernel, m=M, bl=bl),
        grid=(N // bl,),
        in_specs=[pl.BlockSpec((M, K), lambda l: (0, 0)),
                  pl.BlockSpec((K, bl), lambda l: (0, l))],
        out_specs=[pl.BlockSpec((M, bl), lambda l: (0, l)),
                   pl.BlockSpec((M, bl), lambda l: (0, l))],
        out_shape=[oshape, oshape],
        compiler_params=_cparams(1),
        name="mm2_eye",
        interpret=_INTERPRET,
    )(A, B)


def kernel(x, W1, W2, W3):
    z1 = _gemv(x, W1, 512, relu=False)
    z2 = _gemv(z1, W2, 512, relu=True)
    out = _gemv(z2, W3, 512, relu=True)

    W1T, A1b = _transpose(W1, z=z1)
    W2T, W2Tb = _transpose(W2)
    W3T, W3Tb = _transpose(W3)

    T1s, D1, D2 = _mm1(A1b, W2Tb, z1, z2)
    DJM, EYE = _mm2(T1s, W3Tb)

    return (out, DJM, W1T, D1, W2T, D2, W3T, EYE)


# speedup vs baseline: 1.5075x; 1.2118x over previous
import functools

import jax
import jax.numpy as jnp
from jax.experimental import pallas as pl
from jax.experimental.pallas import tpu as pltpu

F32 = jnp.float32
BF16 = jnp.bfloat16
_VMEM_LIMIT = 56 * 1024 * 1024
_INTERPRET = False


def _cparams(n):
    return pltpu.CompilerParams(
        dimension_semantics=("arbitrary",) * n,
        vmem_limit_bytes=_VMEM_LIMIT,
    )


def _gemv_trans_kernel(h_ref, w_ref, z_ref, wt_ref, wtb_ref, *, relu, scale):
    h = h_ref[...]
    if relu:
        h = jnp.maximum(h, 0.0)
    z_ref[...] = jax.lax.dot_general(
        h, w_ref[...], (((1,), (1,)), ((), ())),
        preferred_element_type=F32)
    w = w_ref[...]
    wt_ref[...] = w.T
    if scale:
        w = w * (h_ref[...] > 0).astype(F32)
    wtb_ref[...] = w.T.astype(BF16)


def _gemv_trans(h, W, bj, relu, scale):
    J, K = W.shape
    return pl.pallas_call(
        functools.partial(_gemv_trans_kernel, relu=relu, scale=scale),
        grid=(J // bj,),
        in_specs=[pl.BlockSpec((1, K), lambda j: (0, 0)),
                  pl.BlockSpec((bj, K), lambda j: (j, 0))],
        out_specs=[pl.BlockSpec((1, bj), lambda j: (0, j)),
                   pl.BlockSpec((K, bj), lambda j: (0, j)),
                   pl.BlockSpec((K, bj), lambda j: (0, j))],
        out_shape=[jax.ShapeDtypeStruct((1, J), F32),
                   jax.ShapeDtypeStruct((K, J), F32),
                   jax.ShapeDtypeStruct((K, J), BF16)],
        compiler_params=_cparams(1),
        name="gemv_trans",
        interpret=_INTERPRET,
    )(h, W)


def _mm1_kernel(a_ref, b_ref, z1_ref, z2_ref, t1_ref, d1_ref, d2_ref,
                *, bi, bj, nblk):
    i = pl.program_id(0)
    j = pl.program_id(1)
    o = jnp.dot(a_ref[...], b_ref[...], preferred_element_type=F32)
    m2 = (z2_ref[...] > 0).astype(F32)
    t1_ref[...] = (o * m2).astype(BF16)
    @pl.when(j // nblk == i)
    def _():
        rows = jax.lax.broadcasted_iota(jnp.int32, (2 * bi, bj), 0) + i * 2 * bi
        cols = jax.lax.broadcasted_iota(jnp.int32, (2 * bi, bj), 1) + j * bj
        eq = rows == cols
        d1_ref[...] = jnp.where(eq, (z1_ref[...] > 0).astype(F32), 0.0)
        d2_ref[...] = jnp.where(eq, m2, 0.0)

    @pl.when(j // nblk != i)
    def _():
        d1_ref[...] = jnp.zeros((2 * bi, bj), F32)
        d2_ref[...] = jnp.zeros((2 * bi, bj), F32)


def _mm1(A, B, z1, z2, bi=1024, bj=256):
    M, K = A.shape
    _, N = B.shape
    nblk = (2 * bi) // bj
    dshape = jax.ShapeDtypeStruct((K, N), F32)
    return pl.pallas_call(
        functools.partial(_mm1_kernel, bi=bi, bj=bj, nblk=nblk),
        grid=(M // bi, N // bj),
        in_specs=[pl.BlockSpec((bi, K), lambda i, j: (i, 0)),
                  pl.BlockSpec((K, bj), lambda i, j: (0, j)),
                  pl.BlockSpec((1, bj), lambda i, j: (0, j)),
                  pl.BlockSpec((1, bj), lambda i, j: (0, j))],
        out_specs=[pl.BlockSpec((bi, bj), lambda i, j: (i, j)),
                   pl.BlockSpec((2 * bi, bj), lambda i, j: (i, j)),
                   pl.BlockSpec((2 * bi, bj), lambda i, j: (i, j))],
        out_shape=[jax.ShapeDtypeStruct((M, N), BF16), dshape, dshape],
        compiler_params=_cparams(2),
        name="mm1_diag",
        interpret=_INTERPRET,
    )(A, B, z1, z2)


def _mm2_kernel(a_ref, b_ref, o_ref, eye_ref, *, m, bl):
    l = pl.program_id(0)
    o_ref[...] = jnp.dot(a_ref[...], b_ref[...], preferred_element_type=F32)
    rows = jax.lax.broadcasted_iota(jnp.int32, (m, bl), 0)
    cols = jax.lax.broadcasted_iota(jnp.int32, (m, bl), 1) + l * bl
    eye_ref[...] = jnp.where(rows == cols, 1.0, 0.0).astype(F32)


def _mm2(A, B, bl=256):
    M, K = A.shape
    _, N = B.shape
    oshape = jax.ShapeDtypeStruct((M, N), F32)
    return pl.pallas_call(
        functools.partial(_mm2_kernel, m=M, bl=bl),
        grid=(N // bl,),
        in_specs=[pl.BlockSpec((M, K), lambda l: (0, 0)),
                  pl.BlockSpec((K, bl), lambda l: (0, l))],
        out_specs=[pl.BlockSpec((M, bl), lambda l: (0, l)),
                   pl.BlockSpec((M, bl), lambda l: (0, l))],
        out_shape=[oshape, oshape],
        compiler_params=_cparams(1),
        name="mm2_eye",
        interpret=_INTERPRET,
    )(A, B)


def kernel(x, W1, W2, W3):
    z1, W1T, W1Tb = _gemv_trans(x, W1, 1024, relu=False, scale=False)
    z2, W2T, W2Tsb = _gemv_trans(z1, W2, 512, relu=True, scale=True)
    out, W3T, W3Tb = _gemv_trans(z2, W3, 512, relu=True, scale=False)

    T1s, D1, D2 = _mm1(W1Tb, W2Tsb, z1, z2)
    DJM, EYE = _mm2(T1s, W3Tb)

    return (out, DJM, W1T, D1, W2T, D2, W3T, EYE)


# speedup vs baseline: 1.5294x; 1.0145x over previous
import functools

import jax
import jax.numpy as jnp
from jax.experimental import pallas as pl
from jax.experimental.pallas import tpu as pltpu

F32 = jnp.float32
BF16 = jnp.bfloat16
_VMEM_LIMIT = 63 * 1024 * 1024
_INTERPRET = False


def _cparams(n):
    return pltpu.CompilerParams(
        dimension_semantics=("arbitrary",) * n,
        vmem_limit_bytes=_VMEM_LIMIT,
    )


def _gemv_trans_kernel(h_ref, w_ref, z_ref, wt_ref, wtb_ref, *eye_ref,
                       relu, scale, eye_b):
    h = h_ref[...]
    if relu:
        h = jnp.maximum(h, 0.0)
    z_ref[...] = jax.lax.dot_general(
        h, w_ref[...], (((1,), (1,)), ((), ())),
        preferred_element_type=F32)
    w = w_ref[...]
    wt_ref[...] = w.T
    if scale:
        w = w * (h_ref[...] > 0).astype(F32)
    wtb_ref[...] = w.T.astype(BF16)
    if eye_b:
        jj = pl.program_id(0)
        rows = jax.lax.broadcasted_iota(jnp.int32, (2048, eye_b), 0)
        cols = jax.lax.broadcasted_iota(jnp.int32, (2048, eye_b), 1) + jj * eye_b
        eye_ref[0][...] = jnp.where(rows == cols, 1.0, 0.0).astype(F32)


def _gemv_trans(h, W, bj, relu, scale, eye=False):
    J, K = W.shape
    eye_b = (2048 * bj) // J if eye else 0
    out_specs = [pl.BlockSpec((1, bj), lambda j: (0, j)),
                 pl.BlockSpec((K, bj), lambda j: (0, j)),
                 pl.BlockSpec((K, bj), lambda j: (0, j))]
    out_shape = [jax.ShapeDtypeStruct((1, J), F32),
                 jax.ShapeDtypeStruct((K, J), F32),
                 jax.ShapeDtypeStruct((K, J), BF16)]
    if eye:
        out_specs.append(pl.BlockSpec((2048, eye_b), lambda j: (0, j)))
        out_shape.append(jax.ShapeDtypeStruct((2048, 2048), F32))
    return pl.pallas_call(
        functools.partial(_gemv_trans_kernel, relu=relu, scale=scale,
                          eye_b=eye_b),
        grid=(J // bj,),
        in_specs=[pl.BlockSpec((1, K), lambda j: (0, 0)),
                  pl.BlockSpec((bj, K), lambda j: (j, 0))],
        out_specs=out_specs,
        out_shape=out_shape,
        compiler_params=_cparams(1),
        name="gemv_trans",
        interpret=_INTERPRET,
    )(h, W)


def _mmf_kernel(a_ref, b2_ref, b3_ref, z1_ref, z2_ref,
                d1_ref, d2_ref, djm_ref, t1s_ref, *, bj):
    j = pl.program_id(0)

    @pl.when(j < 32)
    def _():
        i = j // 16
        jj = j - i * 16
        o = jnp.dot(a_ref[...], b2_ref[...], preferred_element_type=F32)
        m2 = (z2_ref[...] > 0).astype(F32)
        r0 = pl.multiple_of(i * 1024, 1024)
        c0 = pl.multiple_of(jj * bj, bj)
        t1s_ref[pl.ds(r0, 1024), pl.ds(c0, bj)] = (o * m2).astype(BF16)

        @pl.when(jj // 8 == i)
        def _():
            rows = jax.lax.broadcasted_iota(jnp.int32, (2048, bj), 0) + i * 2048
            cols = jax.lax.broadcasted_iota(jnp.int32, (2048, bj), 1) + jj * bj
            eq = rows == cols
            d1_ref[...] = jnp.where(eq, (z1_ref[...] > 0).astype(F32), 0.0)
            d2_ref[...] = jnp.where(eq, m2, 0.0)

        @pl.when(jj // 8 != i)
        def _():
            d1_ref[...] = jnp.zeros((2048, bj), F32)
            d2_ref[...] = jnp.zeros((2048, bj), F32)

    @pl.when(j >= 32)
    def _():
        l = j - 32
        del l
        djm_ref[...] = jnp.dot(t1s_ref[...], b3_ref[...],
                               preferred_element_type=F32)


def _mm_fused(A, B2, B3, z1, z2, bj=256):
    d_shape = jax.ShapeDtypeStruct((4096, 4096), F32)
    o_shape = jax.ShapeDtypeStruct((2048, 2048), F32)
    j16 = lambda j: jnp.where(j < 32, j % 16, 15)
    return pl.pallas_call(
        functools.partial(_mmf_kernel, bj=bj),
        grid=(40,),
        in_specs=[
            pl.BlockSpec((1024, 4096), lambda j: (jnp.minimum(j // 16, 1), 0)),
            pl.BlockSpec((4096, bj), lambda j: (0, j16(j))),
            pl.BlockSpec((4096, bj), lambda j: (0, jnp.clip(j - 32, 0, 7))),
            pl.BlockSpec((1, bj), lambda j: (0, j16(j))),
            pl.BlockSpec((1, bj), lambda j: (0, j16(j))),
        ],
        out_specs=[
            pl.BlockSpec((2048, bj), lambda j: (jnp.minimum(j // 16, 1), j16(j))),
            pl.BlockSpec((2048, bj), lambda j: (jnp.minimum(j // 16, 1), j16(j))),
            pl.BlockSpec((2048, bj), lambda j: (0, jnp.clip(j - 32, 0, 7))),
        ],
        out_shape=[d_shape, d_shape, o_shape],
        scratch_shapes=[pltpu.VMEM((2048, 4096), BF16)],
        compiler_params=_cparams(1),
        name="mm_fused",
        interpret=_INTERPRET,
    )(A, B2, B3, z1, z2)


def _mm1_kernel(a_ref, b_ref, z1_ref, z2_ref, t1_ref, d1_ref, d2_ref,
                *, bi, bj, nblk):
    i = pl.program_id(0)
    j = pl.program_id(1)
    o = jnp.dot(a_ref[...], b_ref[...], preferred_element_type=F32)
    m2 = (z2_ref[...] > 0).astype(F32)
    t1_ref[...] = (o * m2).astype(BF16)
    @pl.when(j // nblk == i)
    def _():
        rows = jax.lax.broadcasted_iota(jnp.int32, (2 * bi, bj), 0) + i * 2 * bi
        cols = jax.lax.broadcasted_iota(jnp.int32, (2 * bi, bj), 1) + j * bj
        eq = rows == cols
        d1_ref[...] = jnp.where(eq, (z1_ref[...] > 0).astype(F32), 0.0)
        d2_ref[...] = jnp.where(eq, m2, 0.0)

    @pl.when(j // nblk != i)
    def _():
        d1_ref[...] = jnp.zeros((2 * bi, bj), F32)
        d2_ref[...] = jnp.zeros((2 * bi, bj), F32)


def _mm1(A, B, z1, z2, bi=1024, bj=256):
    M, K = A.shape
    _, N = B.shape
    nblk = (2 * bi) // bj
    dshape = jax.ShapeDtypeStruct((K, N), F32)
    return pl.pallas_call(
        functools.partial(_mm1_kernel, bi=bi, bj=bj, nblk=nblk),
        grid=(M // bi, N // bj),
        in_specs=[pl.BlockSpec((bi, K), lambda i, j: (i, 0)),
                  pl.BlockSpec((K, bj), lambda i, j: (0, j)),
                  pl.BlockSpec((1, bj), lambda i, j: (0, j)),
                  pl.BlockSpec((1, bj), lambda i, j: (0, j))],
        out_specs=[pl.BlockSpec((bi, bj), lambda i, j: (i, j)),
                   pl.BlockSpec((2 * bi, bj), lambda i, j: (i, j)),
                   pl.BlockSpec((2 * bi, bj), lambda i, j: (i, j))],
        out_shape=[jax.ShapeDtypeStruct((M, N), BF16), dshape, dshape],
        compiler_params=_cparams(2),
        name="mm1_diag",
        interpret=_INTERPRET,
    )(A, B, z1, z2)


def _mm2_kernel(a_ref, b_ref, o_ref, eye_ref, *, m, bl):
    l = pl.program_id(0)
    o_ref[...] = jnp.dot(a_ref[...], b_ref[...], preferred_element_type=F32)
    rows = jax.lax.broadcasted_iota(jnp.int32, (m, bl), 0)
    cols = jax.lax.broadcasted_iota(jnp.int32, (m, bl), 1) + l * bl
    eye_ref[...] = jnp.where(rows == cols, 1.0, 0.0).astype(F32)


def _mm2(A, B, bl=256):
    M, K = A.shape
    _, N = B.shape
    oshape = jax.ShapeDtypeStruct((M, N), F32)
    return pl.pallas_call(
        functools.partial(_mm2_kernel, m=M, bl=bl),
        grid=(N // bl,),
        in_specs=[pl.BlockSpec((M, K), lambda l: (0, 0)),
                  pl.BlockSpec((K, bl), lambda l: (0, l))],
        out_specs=[pl.BlockSpec((M, bl), lambda l: (0, l)),
                   pl.BlockSpec((M, bl), lambda l: (0, l))],
        out_shape=[oshape, oshape],
        compiler_params=_cparams(1),
        name="mm2_eye",
        interpret=_INTERPRET,
    )(A, B)


def kernel(x, W1, W2, W3):
    z1, W1T, W1Tb = _gemv_trans(x, W1, 1024, relu=False, scale=False)
    z2, W2T, W2Tsb = _gemv_trans(z1, W2, 512, relu=True, scale=True)
    out, W3T, W3Tb, EYE = _gemv_trans(z2, W3, 512, relu=True, scale=False,
                                      eye=True)

    D1, D2, DJM = _mm_fused(W1Tb, W2Tsb, W3Tb, z1, z2)

    return (out, DJM, W1T, D1, W2T, D2, W3T, EYE)
